# baseline (speedup 1.0000x reference)
import functools

import jax
import jax.numpy as jnp
from jax import lax
from jax.experimental import pallas as pl
from jax.experimental.pallas import tpu as pltpu
from jax.experimental.pallas import tpu_sc as plsc


def _make_topk(B, M, U, C, interpret=False):
    def body(q_ref, k_ref, o_ref):
        kk = k_ref[0]
        rk = 1.0 / (jnp.sqrt(jnp.sum(kk * kk, axis=1, keepdims=True)) + 1e-8)
        kn = kk * rk
        q = q_ref[0]
        rq = 1.0 / (jnp.sqrt(jnp.sum(q * q, axis=1, keepdims=True)) + 1e-8)
        qn = q * rq
        sim = lax.dot_general(
            qn.astype(jnp.bfloat16), kn.astype(jnp.bfloat16),
            (((1,), (1,)), ((), ())),
            preferred_element_type=jnp.float32)
        idx = jnp.argmax(sim, axis=1).astype(jnp.int32)
        o_ref[0, 0] = idx + pl.program_id(0) * U

    return pl.pallas_call(
        body,
        grid=(B,),
        in_specs=[
            pl.BlockSpec((1, M, C), lambda b: (b, 0, 0)),
            pl.BlockSpec((1, U, C), lambda b: (b, 0, 0)),
        ],
        out_specs=pl.BlockSpec((1, 1, M), lambda b: (b, 0, 0)),
        out_shape=jax.ShapeDtypeStruct((B, 1, M), jnp.int32),
        interpret=interpret,
    )


def _gather(k2, idxflat):
    R = idxflat.shape[0]
    C = k2.shape[1]
    info = plsc.get_sparse_core_info()
    NC, NS = info.num_cores, info.num_subcores
    NW = NC * NS
    rpw = R // NW
    mesh = plsc.VectorSubcoreMesh(core_axis_name="c", subcore_axis_name="s")

    @functools.partial(
        pl.kernel, mesh=mesh,
        out_type=jax.ShapeDtypeStruct((R, C), jnp.float32),
        scratch_types=[
            pltpu.VMEM((rpw,), jnp.int32),
            pltpu.VMEM((rpw, C), jnp.float32),
            pltpu.SemaphoreType.DMA,
        ],
    )
    def sc_fn(k_hbm, idx_hbm, out_hbm, idx_v, rows_v, sem):
        wid = lax.axis_index("s") * NC + lax.axis_index("c")
        base = wid * rpw
        pltpu.sync_copy(idx_hbm.at[pl.ds(base, rpw)], idx_v)
        pltpu.async_copy(k_hbm.at[idx_v], rows_v, sem).wait()
        pltpu.sync_copy(rows_v, out_hbm.at[pl.ds(base, rpw)])

    return sc_fn(k2, idxflat)


def _make_assemble(B, C, HW, M, W, h0, h1, w0, w1):
    wq = w1 - w0

    def body(x_ref, r_ref, o_ref):
        o_ref[0] = x_ref[0]
        rT = jnp.transpose(r_ref[0])
        for j in range(h1 - h0):
            dst = (h0 + j) * W + w0
            o_ref[0, :, dst:dst + wq] = (
                rT[:, j * wq:(j + 1) * wq]
                + x_ref[0][:, dst:dst + wq]) * 0.5

    return pl.pallas_call(
        body,
        grid=(B,),
        in_specs=[pl.BlockSpec((1, C, HW), lambda b: (b, 0, 0)),
                  pl.BlockSpec((1, M, C), lambda b: (b, 0, 0))],
        out_specs=pl.BlockSpec((1, C, HW), lambda b: (b, 0, 0)),
        out_shape=jax.ShapeDtypeStruct((B, C, HW), jnp.float32),
    )


def kernel(input):
    x = input
    B, C, H, W = x.shape
    h0, h1 = H // 4, 3 * H // 4
    w0, w1 = W // 4, 3 * W // 4
    M = (h1 - h0) * (w1 - w0)
    U = H * W - M

    q_cm = x[:, :, h0:h1, w0:w1].reshape(B, C, M)
    top = x[:, :, :h0, :].reshape(B, C, -1)
    mid = jnp.concatenate(
        [x[:, :, h0:h1, :w0], x[:, :, h0:h1, w1:]], axis=3).reshape(B, C, -1)
    bot = x[:, :, h1:, :].reshape(B, C, -1)
    k_cm = jnp.concatenate([top, mid, bot], axis=2)

    q = q_cm.transpose(0, 2, 1)
    k = k_cm.transpose(0, 2, 1)

    idx = _make_topk(B, M, U, C)(q, k)
    retrieved = _gather(k.reshape(B * U, C), idx.reshape(B * M))

    HW = H * W
    out = _make_assemble(B, C, HW, M, W, h0, h1, w0, w1)(
        x.reshape(B, C, HW), retrieved.reshape(B, M, C))
    return out.reshape(B, C, H, W)


# speedup vs baseline: 1.4277x; 1.4277x over previous
import functools

import jax
import jax.numpy as jnp
from jax import lax
from jax.experimental import pallas as pl
from jax.experimental.pallas import tpu as pltpu
from jax.experimental.pallas import tpu_sc as plsc


def _make_topk(B, M, U, C, interpret=False):
    def body(q_ref, k_ref, o_ref):
        kk = k_ref[0]
        kn = kk / (jnp.sqrt(jnp.sum(kk * kk, axis=1, keepdims=True)) + 1e-8)
        q = q_ref[0]
        qn = q / (jnp.sqrt(jnp.sum(q * q, axis=1, keepdims=True)) + 1e-8)
        sim = lax.dot_general(
            qn.astype(jnp.bfloat16), kn.astype(jnp.bfloat16),
            (((1,), (1,)), ((), ())),
            preferred_element_type=jnp.float32)
        idx = jnp.argmax(sim, axis=1).astype(jnp.int32)
        o_ref[0, 0] = idx + pl.program_id(0) * U

    return pl.pallas_call(
        body,
        grid=(B,),
        in_specs=[
            pl.BlockSpec((1, M, C), lambda b: (b, 0, 0)),
            pl.BlockSpec((1, U, C), lambda b: (b, 0, 0)),
        ],
        out_specs=pl.BlockSpec((1, 1, M), lambda b: (b, 0, 0)),
        out_shape=jax.ShapeDtypeStruct((B, 1, M), jnp.int32),
        interpret=interpret,
    )


def _gather_blend(k2, q2, idxflat):
    R, C = q2.shape
    info = plsc.get_sparse_core_info()
    NC, NS = info.num_cores, info.num_subcores
    NW = NC * NS
    rpw = R // NW
    mesh = plsc.VectorSubcoreMesh(core_axis_name="c", subcore_axis_name="s")

    @functools.partial(
        pl.kernel, mesh=mesh,
        out_type=jax.ShapeDtypeStruct((R, C), jnp.float32),
        scratch_types=[
            pltpu.VMEM((rpw,), jnp.int32),
            pltpu.VMEM((rpw, C), jnp.float32),
            pltpu.VMEM((rpw, C), jnp.float32),
            pltpu.SemaphoreType.DMA,
        ],
    )
    def sc_fn(k_hbm, q_hbm, idx_hbm, out_hbm, idx_v, rows_v, q_v, sem):
        wid = lax.axis_index("s") * NC + lax.axis_index("c")
        base = wid * rpw
        pltpu.sync_copy(idx_hbm.at[pl.ds(base, rpw)], idx_v)
        cp = pltpu.async_copy(k_hbm.at[idx_v], rows_v, sem)
        pltpu.sync_copy(q_hbm.at[pl.ds(base, rpw)], q_v)
        cp.wait()

        def row(r, carry):
            for c in range(0, C, 16):
                s = pl.ds(c, 16)
                rows_v[r, s] = (rows_v[r, s] + q_v[r, s]) * 0.5
            return carry

        lax.fori_loop(0, rpw, row, 0)
        pltpu.sync_copy(rows_v, out_hbm.at[pl.ds(base, rpw)])

    return sc_fn(k2, q2, idxflat)


def kernel(input):
    x = input
    B, C, H, W = x.shape
    h0, h1 = H // 4, 3 * H // 4
    w0, w1 = W // 4, 3 * W // 4
    M = (h1 - h0) * (w1 - w0)
    U = H * W - M

    q_cm = x[:, :, h0:h1, w0:w1].reshape(B, C, M)
    top = x[:, :, :h0, :].reshape(B, C, -1)
    mid = jnp.concatenate(
        [x[:, :, h0:h1, :w0], x[:, :, h0:h1, w1:]], axis=3).reshape(B, C, -1)
    bot = x[:, :, h1:, :].reshape(B, C, -1)
    k_cm = jnp.concatenate([top, mid, bot], axis=2)

    q = q_cm.transpose(0, 2, 1)
    k = k_cm.transpose(0, 2, 1)

    idx = _make_topk(B, M, U, C)(q, k)
    blended = _gather_blend(
        k.reshape(B * U, C), q.reshape(B * M, C), idx.reshape(B * M))

    patch = blended.reshape(B, h1 - h0, w1 - w0, C).transpose(0, 3, 1, 2)
    return x.at[:, :, h0:h1, w0:w1].set(patch)


# speedup vs baseline: 1.4294x; 1.0012x over previous
import functools

import jax
import jax.numpy as jnp
from jax import lax
from jax.experimental import pallas as pl
from jax.experimental.pallas import tpu as pltpu
from jax.experimental.pallas import tpu_sc as plsc


def _make_topk(B, M, U, C):
    def body(q_ref, k_ref, o_ref):
        kk = k_ref[0]
        kn = kk / (jnp.sqrt(jnp.sum(kk * kk, axis=1, keepdims=True)) + 1e-8)
        q = q_ref[0]
        qn = q / (jnp.sqrt(jnp.sum(q * q, axis=1, keepdims=True)) + 1e-8)
        sim = lax.dot_general(
            qn.astype(jnp.bfloat16), kn.astype(jnp.bfloat16),
            (((1,), (1,)), ((), ())),
            preferred_element_type=jnp.float32)
        idx = jnp.argmax(sim, axis=1).astype(jnp.int32)
        o_ref[0, 0] = idx + pl.program_id(0) * U

    return pl.pallas_call(
        body,
        grid=(B,),
        in_specs=[
            pl.BlockSpec((1, M, C), lambda b: (b, 0, 0)),
            pl.BlockSpec((1, U, C), lambda b: (b, 0, 0)),
        ],
        out_specs=pl.BlockSpec((1, 1, M), lambda b: (b, 0, 0)),
        out_shape=jax.ShapeDtypeStruct((B, 1, M), jnp.int32),
    )


def _gather_blend(k2, q2, idxflat):
    R, C = q2.shape
    info = plsc.get_sparse_core_info()
    NC, NS = info.num_cores, info.num_subcores
    NW = NC * NS
    rpw = R // NW
    mesh = plsc.VectorSubcoreMesh(core_axis_name="c", subcore_axis_name="s")

    @functools.partial(
        pl.kernel, mesh=mesh,
        out_type=jax.ShapeDtypeStruct((R, C), jnp.float32),
        scratch_types=[
            pltpu.VMEM((rpw,), jnp.int32),
            pltpu.VMEM((rpw, C), jnp.float32),
            pltpu.VMEM((rpw, C), jnp.float32),
            pltpu.SemaphoreType.DMA,
        ],
    )
    def sc_fn(k_hbm, q_hbm, idx_hbm, out_hbm, idx_v, rows_v, q_v, sem):
        wid = lax.axis_index("s") * NC + lax.axis_index("c")
        base = wid * rpw
        pltpu.sync_copy(idx_hbm.at[pl.ds(base, rpw)], idx_v)
        cp = pltpu.async_copy(k_hbm.at[idx_v], rows_v, sem)
        pltpu.sync_copy(q_hbm.at[pl.ds(base, rpw)], q_v)
        cp.wait()

        def row(r, carry):
            for c in range(0, C, 16):
                s = pl.ds(c, 16)
                rows_v[r, s] = (rows_v[r, s] + q_v[r, s]) * 0.5
            return carry

        lax.fori_loop(0, rpw, row, 0)
        pltpu.sync_copy(rows_v, out_hbm.at[pl.ds(base, rpw)])

    return sc_fn(k2, q2, idxflat)


def kernel(input):
    x = input
    B, C, H, W = x.shape
    h0, h1 = H // 4, 3 * H // 4
    w0, w1 = W // 4, 3 * W // 4
    M = (h1 - h0) * (w1 - w0)
    U = H * W - M

    q_cm = x[:, :, h0:h1, w0:w1].reshape(B, C, M)
    top = x[:, :, :h0, :].reshape(B, C, -1)
    mid = jnp.concatenate(
        [x[:, :, h0:h1, :w0], x[:, :, h0:h1, w1:]], axis=3).reshape(B, C, -1)
    bot = x[:, :, h1:, :].reshape(B, C, -1)
    k_cm = jnp.concatenate([top, mid, bot], axis=2)

    q = q_cm.transpose(0, 2, 1)
    k = k_cm.transpose(0, 2, 1)

    idx = _make_topk(B, M, U, C)(q, k)
    blended = _gather_blend(
        k.reshape(B * U, C), q.reshape(B * M, C), idx.reshape(B * M))

    patch = blended.reshape(B, h1 - h0, w1 - w0, C).transpose(0, 3, 1, 2)
    return x.at[:, :, h0:h1, w0:w1].set(patch)
